# baseline (speedup 1.0000x reference)
import numpy as np

import jax
import jax.numpy as jnp
from jax import lax
from jax.experimental import pallas as pl
from jax.experimental.pallas import tpu as pltpu
from jax.experimental.pallas import tpu_sc as plsc

NC = 2
NS = 16
LANES = 16
NW = NC * NS

B = 4096
H = 50
D = 64
BPW = B // NW
ROWS2 = BPW // 2
EPS = 1e-5

K = 10
NGRP = (H + K - 1) // K

_PERM = np.concatenate([np.arange(0, 16), np.arange(32, 48),
                        np.arange(16, 32), np.arange(48, 64)])


def _grp_hs(g):
    return range(g * K, min((g + 1) * K, H))


def _pool_body(xt_hbm, user_hbm, genre_hbm, out_hbm,
               idx_v, bufs, acc, sem0, sem1):
    wid = lax.axis_index("s") * NC + lax.axis_index("c")
    sems = (sem0, sem1)
    mask = jnp.broadcast_to(jnp.uint32(0xFFFF0000), (LANES,))
    sh16 = jnp.broadcast_to(jnp.uint32(16), (LANES,))

    for t, table in ((0, user_hbm), (1, genre_hbm)):
        pltpu.sync_copy(xt_hbm.at[t, :, wid], idx_v)

        def _issue(g):
            bank = g % 2
            for k, h in enumerate(_grp_hs(g)):
                pltpu.async_copy(table.at[idx_v.at[h]], bufs.at[bank, k],
                                 sems[bank])

        def _drain(g):
            bank = g % 2
            for k, h in enumerate(_grp_hs(g)):
                pltpu.make_async_copy(table.at[idx_v.at[h]], bufs.at[bank, k],
                                      sems[bank]).wait()

        def _accum(g):
            bank = g % 2
            nk = len(_grp_hs(g))

            @plsc.parallel_loop(0, ROWS2, unroll=2)
            def body(p):
                for half in range(2):
                    i = 2 * p + half
                    for c in range(2):
                        s_e = s_o = None
                        for k in range(nk):
                            u = bufs[bank, k, i, pl.ds(c * LANES, LANES)]
                            lo = plsc.bitcast(u << sh16, jnp.float32)
                            hi = plsc.bitcast(u & mask, jnp.float32)
                            s_e = lo if s_e is None else s_e + lo
                            s_o = hi if s_o is None else s_o + hi
                        base = half * 64 + c * 32
                        if g > 0:
                            s_e = s_e + acc[p, pl.ds(base, LANES)]
                            s_o = s_o + acc[p, pl.ds(base + LANES, LANES)]
                        acc[p, pl.ds(base, LANES)] = s_e
                        acc[p, pl.ds(base + LANES, LANES)] = s_o

        _issue(0)
        _issue(1)
        for g in range(NGRP):
            _drain(g)
            _accum(g)
            if g + 2 < NGRP:
                _issue(g + 2)

        pltpu.sync_copy(acc, out_hbm.at[t, pl.ds(wid * ROWS2, ROWS2)])


def _pool(xt, user_packed, genre_packed):
    mesh = plsc.VectorSubcoreMesh(core_axis_name="c", subcore_axis_name="s",
                                  num_cores=NC, num_subcores=NS)
    return pl.kernel(
        _pool_body,
        out_type=jax.ShapeDtypeStruct((2, B // 2, 128), jnp.float32),
        mesh=mesh,
        scratch_types=[
            pltpu.VMEM((H, BPW), jnp.int32),
            pltpu.VMEM((2, K, BPW, 32), jnp.uint32),
            pltpu.VMEM((ROWS2, 128), jnp.float32),
            pltpu.SemaphoreType.DMA,
            pltpu.SemaphoreType.DMA,
        ],
        compiler_params=pltpu.CompilerParams(use_tc_tiling_on_sc=False,
                                             needs_layout_passes=False),
    )(xt, user_packed, genre_packed)


def _bn_dot_body(emb_ref, gamma_ref, beta_ref, out_ref):
    gamma = gamma_ref[...]
    beta = beta_ref[...]

    def bn(h):
        n = 2.0 / B
        m = jnp.sum(h, axis=0, keepdims=True) * n
        sq = jnp.sum(h * h, axis=0, keepdims=True) * n
        mu = (m[:, :D] + m[:, D:]) * 0.5
        var = (sq[:, :D] + sq[:, D:]) * 0.5 - mu * mu
        a = gamma * lax.rsqrt(var + EPS)
        b = beta - a * mu
        a2 = jnp.concatenate([a, a], axis=1)
        b2 = jnp.concatenate([b, b], axis=1)
        return h * a2 + b2

    u = bn(emb_ref[0] * (1.0 / H))
    g = bn(emb_ref[1] * (1.0 / H))
    prod = u * g
    z0 = jnp.sum(prod[:, :D], axis=1, keepdims=True)
    z1 = jnp.sum(prod[:, D:], axis=1, keepdims=True)
    out_ref[...] = jax.nn.sigmoid(jnp.concatenate([z0, z1], axis=1))


def _bn_dot(pooled, gamma, beta):
    return pl.pallas_call(
        _bn_dot_body,
        out_shape=jax.ShapeDtypeStruct((B // 2, 2), jnp.float32),
    )(pooled, gamma, beta)


def _pack_u32(table):
    u = lax.bitcast_convert_type(table.T, jnp.uint32)
    r = u + jnp.uint32(0x7FFF) + ((u >> 16) & jnp.uint32(1))
    top = r & jnp.uint32(0xFFFF0000)
    pT = lax.optimization_barrier((top[:32] >> 16) | top[32:])
    return pT.T


def kernel(x, user_table, genre_table, gamma, beta):
    xt = jnp.transpose(x.astype(jnp.int32), (0, 2, 1)).reshape(2, H, NW, 128)
    pooled = _pool(xt, _pack_u32(user_table), _pack_u32(genre_table))
    perm = jnp.asarray(_PERM)
    z = _bn_dot(pooled, gamma[perm].reshape(1, D), beta[perm].reshape(1, D))
    return z.reshape(B)


# speedup vs baseline: 1.5830x; 1.5830x over previous
import numpy as np

import jax
import jax.numpy as jnp
from jax import lax
from jax.experimental import pallas as pl
from jax.experimental.pallas import tpu as pltpu
from jax.experimental.pallas import tpu_sc as plsc

NC = 2
NS = 16
LANES = 16
NW = NC * NS

B = 4096
H = 50
D = 64
BPW = B // NW
ROWS2 = BPW // 2
EPS = 1e-5

K = 10
NGRP = (H + K - 1) // K

_PERM = np.concatenate([np.arange(0, 16), np.arange(32, 48),
                        np.arange(16, 32), np.arange(48, 64)])


def _grp_hs(g):
    return range(g * K, min((g + 1) * K, H))


def _pool_body(xt_hbm, packed_hbm, out_hbm,
               idx_v, bufs, acc, sem0, sem1):
    wid = lax.axis_index("s") * NC + lax.axis_index("c")
    sems = (sem0, sem1)
    mask = jnp.broadcast_to(jnp.uint32(0xFFFF0000), (LANES,))
    sh16 = jnp.broadcast_to(jnp.uint32(16), (LANES,))

    for t in (0, 1):
        table = packed_hbm.at[t]
        pltpu.sync_copy(xt_hbm.at[t, :, wid], idx_v)

        def _issue(g):
            bank = g % 2
            for k, h in enumerate(_grp_hs(g)):
                pltpu.async_copy(table.at[idx_v.at[h]], bufs.at[bank, k],
                                 sems[bank])

        def _drain(g):
            bank = g % 2
            for k, h in enumerate(_grp_hs(g)):
                pltpu.make_async_copy(table.at[idx_v.at[h]], bufs.at[bank, k],
                                      sems[bank]).wait()

        def _accum(g):
            bank = g % 2
            nk = len(_grp_hs(g))

            @plsc.parallel_loop(0, ROWS2, unroll=2)
            def body(p):
                for half in range(2):
                    i = 2 * p + half
                    for c in range(2):
                        s_e = s_o = None
                        for k in range(nk):
                            u = bufs[bank, k, i, pl.ds(c * LANES, LANES)]
                            lo = plsc.bitcast(u << sh16, jnp.float32)
                            hi = plsc.bitcast(u & mask, jnp.float32)
                            s_e = lo if s_e is None else s_e + lo
                            s_o = hi if s_o is None else s_o + hi
                        base = half * 64 + c * 32
                        if g > 0:
                            s_e = s_e + acc[p, pl.ds(base, LANES)]
                            s_o = s_o + acc[p, pl.ds(base + LANES, LANES)]
                        acc[p, pl.ds(base, LANES)] = s_e
                        acc[p, pl.ds(base + LANES, LANES)] = s_o

        _issue(0)
        _issue(1)
        for g in range(NGRP):
            _drain(g)
            _accum(g)
            if g + 2 < NGRP:
                _issue(g + 2)

        pltpu.sync_copy(acc, out_hbm.at[t, pl.ds(wid * ROWS2, ROWS2)])


def _pool(xt, packed):
    mesh = plsc.VectorSubcoreMesh(core_axis_name="c", subcore_axis_name="s",
                                  num_cores=NC, num_subcores=NS)
    return pl.kernel(
        _pool_body,
        out_type=jax.ShapeDtypeStruct((2, B // 2, 128), jnp.float32),
        mesh=mesh,
        scratch_types=[
            pltpu.VMEM((H, BPW), jnp.int32),
            pltpu.VMEM((2, K, BPW, 32), jnp.uint32),
            pltpu.VMEM((ROWS2, 128), jnp.float32),
            pltpu.SemaphoreType.DMA,
            pltpu.SemaphoreType.DMA,
        ],
        compiler_params=pltpu.CompilerParams(use_tc_tiling_on_sc=False,
                                             needs_layout_passes=False),
    )(xt, packed)


def _bn_dot_body(emb_ref, gamma_ref, beta_ref, out_ref):
    gamma = gamma_ref[...]
    beta = beta_ref[...]

    def bn(h):
        n = 2.0 / B
        m = jnp.sum(h, axis=0, keepdims=True) * n
        sq = jnp.sum(h * h, axis=0, keepdims=True) * n
        mu = (m[:, :D] + m[:, D:]) * 0.5
        var = (sq[:, :D] + sq[:, D:]) * 0.5 - mu * mu
        a = gamma * lax.rsqrt(var + EPS)
        b = beta - a * mu
        a2 = jnp.concatenate([a, a], axis=1)
        b2 = jnp.concatenate([b, b], axis=1)
        return h * a2 + b2

    u = bn(emb_ref[0] * (1.0 / H))
    g = bn(emb_ref[1] * (1.0 / H))
    prod = u * g
    z0 = jnp.sum(prod[:, :D], axis=1, keepdims=True)
    z1 = jnp.sum(prod[:, D:], axis=1, keepdims=True)
    out_ref[...] = jax.nn.sigmoid(jnp.concatenate([z0, z1], axis=1))


def _bn_dot(pooled, gamma, beta):
    return pl.pallas_call(
        _bn_dot_body,
        out_shape=jax.ShapeDtypeStruct((B // 2, 2), jnp.float32),
    )(pooled, gamma, beta)


def _pack_u32(table):
    u = lax.bitcast_convert_type(table, jnp.uint32)
    r = u + jnp.uint32(0x7FFF) + ((u >> 16) & jnp.uint32(1))
    top = r & jnp.uint32(0xFFFF0000)
    return (top[:, :32] >> 16) | top[:, 32:]


def kernel(x, user_table, genre_table, gamma, beta):
    xt = jnp.transpose(x.astype(jnp.int32), (0, 2, 1)).reshape(2, H, NW, 128)
    packed = lax.optimization_barrier(
        jnp.stack([_pack_u32(user_table), _pack_u32(genre_table)]))
    pooled = _pool(xt, packed)
    perm = jnp.asarray(_PERM)
    z = _bn_dot(pooled, gamma[perm].reshape(1, D), beta[perm].reshape(1, D))
    return z.reshape(B)


# speedup vs baseline: 1.7929x; 1.1326x over previous
import numpy as np

import jax
import jax.numpy as jnp
from jax import lax
from jax.experimental import pallas as pl
from jax.experimental.pallas import tpu as pltpu
from jax.experimental.pallas import tpu_sc as plsc

NC = 2
NS = 16
LANES = 16
NW = NC * NS

B = 4096
H = 50
D = 64
BPW = B // NW
ROWS2 = BPW // 2
EPS = 1e-5

K = 10
NGRP = (H + K - 1) // K

_PERM = np.concatenate([np.arange(0, 16), np.arange(32, 48),
                        np.arange(16, 32), np.arange(48, 64)])


def _grp_hs(g):
    return range(g * K, min((g + 1) * K, H))


def _pool_body(xt_hbm, packed_hbm, out_hbm,
               idx_v, bufs, acc, sem0, sem1):
    wid = lax.axis_index("s") * NC + lax.axis_index("c")
    sems = (sem0, sem1)
    mask = jnp.broadcast_to(jnp.uint32(0xFFFF0000), (LANES,))
    sh16 = jnp.broadcast_to(jnp.uint32(16), (LANES,))

    for t in (0, 1):
        table = packed_hbm.at[pl.ds(t * 100000, 100000)]
        pltpu.sync_copy(xt_hbm.at[t, :, wid], idx_v)

        def _issue(g):
            bank = g % 2
            for k, h in enumerate(_grp_hs(g)):
                pltpu.async_copy(table.at[idx_v.at[h]], bufs.at[bank, k],
                                 sems[bank])

        def _drain(g):
            bank = g % 2
            for k, h in enumerate(_grp_hs(g)):
                pltpu.make_async_copy(table.at[idx_v.at[h]], bufs.at[bank, k],
                                      sems[bank]).wait()

        def _accum(g):
            bank = g % 2
            nk = len(_grp_hs(g))

            @plsc.parallel_loop(0, ROWS2, unroll=2)
            def body(p):
                for half in range(2):
                    i = 2 * p + half
                    for c in range(2):
                        s_e = s_o = None
                        for k in range(nk):
                            u = bufs[bank, k, i, pl.ds(c * LANES, LANES)]
                            lo = plsc.bitcast(u << sh16, jnp.float32)
                            hi = plsc.bitcast(u & mask, jnp.float32)
                            s_e = lo if s_e is None else s_e + lo
                            s_o = hi if s_o is None else s_o + hi
                        base = half * 64 + c * 32
                        if g > 0:
                            s_e = s_e + acc[p, pl.ds(base, LANES)]
                            s_o = s_o + acc[p, pl.ds(base + LANES, LANES)]
                        acc[p, pl.ds(base, LANES)] = s_e
                        acc[p, pl.ds(base + LANES, LANES)] = s_o

        _issue(0)
        _issue(1)
        for g in range(NGRP):
            _drain(g)
            _accum(g)
            if g + 2 < NGRP:
                _issue(g + 2)

        pltpu.sync_copy(acc, out_hbm.at[t, pl.ds(wid * ROWS2, ROWS2)])


def _pool(xt, packed):
    mesh = plsc.VectorSubcoreMesh(core_axis_name="c", subcore_axis_name="s",
                                  num_cores=NC, num_subcores=NS)
    return pl.kernel(
        _pool_body,
        out_type=jax.ShapeDtypeStruct((2, B // 2, 128), jnp.float32),
        mesh=mesh,
        scratch_types=[
            pltpu.VMEM((H, BPW), jnp.int32),
            pltpu.VMEM((2, K, BPW, 32), jnp.uint32),
            pltpu.VMEM((ROWS2, 128), jnp.float32),
            pltpu.SemaphoreType.DMA,
            pltpu.SemaphoreType.DMA,
        ],
        compiler_params=pltpu.CompilerParams(use_tc_tiling_on_sc=False,
                                             needs_layout_passes=False),
    )(xt, packed)


def _bn_dot_body(emb_ref, gamma_ref, beta_ref, out_ref):
    gamma = gamma_ref[...]
    beta = beta_ref[...]

    def bn(h):
        n = 2.0 / B
        m = jnp.sum(h, axis=0, keepdims=True) * n
        sq = jnp.sum(h * h, axis=0, keepdims=True) * n
        mu = (m[:, :D] + m[:, D:]) * 0.5
        var = (sq[:, :D] + sq[:, D:]) * 0.5 - mu * mu
        a = gamma * lax.rsqrt(var + EPS)
        b = beta - a * mu
        a2 = jnp.concatenate([a, a], axis=1)
        b2 = jnp.concatenate([b, b], axis=1)
        return h * a2 + b2

    u = bn(emb_ref[0] * (1.0 / H))
    g = bn(emb_ref[1] * (1.0 / H))
    prod = u * g
    z0 = jnp.sum(prod[:, :D], axis=1, keepdims=True)
    z1 = jnp.sum(prod[:, D:], axis=1, keepdims=True)
    out_ref[...] = jax.nn.sigmoid(jnp.concatenate([z0, z1], axis=1))


def _bn_dot(pooled, gamma, beta):
    return pl.pallas_call(
        _bn_dot_body,
        out_shape=jax.ShapeDtypeStruct((B // 2, 2), jnp.float32),
    )(pooled, gamma, beta)


def _pack_u32(table):
    def bf16top(x):
        u = lax.bitcast_convert_type(x, jnp.uint32)
        r = u + jnp.uint32(0x7FFF) + ((u >> 16) & jnp.uint32(1))
        return r & jnp.uint32(0xFFFF0000)
    return (bf16top(table[:, :32]) >> 16) | bf16top(table[:, 32:])


def kernel(x, user_table, genre_table, gamma, beta):
    xt = jnp.transpose(x.astype(jnp.int32), (0, 2, 1)).reshape(2, H, NW, 128)
    packed = lax.optimization_barrier(
        jnp.concatenate([_pack_u32(user_table), _pack_u32(genre_table)]))
    pooled = _pool(xt, packed)
    perm = jnp.asarray(_PERM)
    z = _bn_dot(pooled, gamma[perm].reshape(1, D), beta[perm].reshape(1, D))
    return z.reshape(B)


# speedup vs baseline: 1.7945x; 1.0009x over previous
import numpy as np

import jax
import jax.numpy as jnp
from jax import lax
from jax.experimental import pallas as pl
from jax.experimental.pallas import tpu as pltpu
from jax.experimental.pallas import tpu_sc as plsc

NC = 2
NS = 16
LANES = 16
NW = NC * NS

B = 4096
H = 50
D = 64
BPW = B // NW
ROWS2 = BPW // 2
EPS = 1e-5

K = 10
NGRP = (H + K - 1) // K

_PERM = np.concatenate([np.arange(0, 16), np.arange(32, 48),
                        np.arange(16, 32), np.arange(48, 64)])


def _grp_hs(g):
    return range(g * K, min((g + 1) * K, H))


def _pool_body(xt_hbm, packed_hbm, out_hbm,
               idx_v, bufs, acc, sem0, sem1):
    wid = lax.axis_index("s") * NC + lax.axis_index("c")
    sems = (sem0, sem1)
    mask = jnp.broadcast_to(jnp.uint32(0xFFFF0000), (LANES,))
    sh16 = jnp.broadcast_to(jnp.uint32(16), (LANES,))

    for t in (0, 1):
        table = packed_hbm.at[pl.ds(t * 100000, 100000)]
        pltpu.sync_copy(xt_hbm.at[t, :, wid], idx_v)

        def _issue(g):
            bank = g % 2
            for k, h in enumerate(_grp_hs(g)):
                pltpu.async_copy(table.at[idx_v.at[h]], bufs.at[bank, k],
                                 sems[bank])

        def _drain(g):
            bank = g % 2
            for k, h in enumerate(_grp_hs(g)):
                pltpu.make_async_copy(table.at[idx_v.at[h]], bufs.at[bank, k],
                                      sems[bank]).wait()

        def _accum(g):
            bank = g % 2
            nk = len(_grp_hs(g))

            @plsc.parallel_loop(0, ROWS2, unroll=2)
            def body(p):
                for half in range(2):
                    i = 2 * p + half
                    for c in range(2):
                        s_e = s_o = None
                        for k in range(nk):
                            u = bufs[bank, k, i, pl.ds(c * LANES, LANES)]
                            lo = plsc.bitcast(u << sh16, jnp.float32)
                            hi = plsc.bitcast(u & mask, jnp.float32)
                            s_e = lo if s_e is None else s_e + lo
                            s_o = hi if s_o is None else s_o + hi
                        base = half * 64 + c * 32
                        if g > 0:
                            s_e = s_e + acc[p, pl.ds(base, LANES)]
                            s_o = s_o + acc[p, pl.ds(base + LANES, LANES)]
                        acc[p, pl.ds(base, LANES)] = s_e
                        acc[p, pl.ds(base + LANES, LANES)] = s_o

        _issue(0)
        _issue(1)
        for g in range(NGRP):
            _drain(g)
            _accum(g)
            if g + 2 < NGRP:
                _issue(g + 2)

        pltpu.sync_copy(acc, out_hbm.at[t, pl.ds(wid * ROWS2, ROWS2)])


def _pool(xt, packed):
    mesh = plsc.VectorSubcoreMesh(core_axis_name="c", subcore_axis_name="s",
                                  num_cores=NC, num_subcores=NS)
    return pl.kernel(
        _pool_body,
        out_type=jax.ShapeDtypeStruct((2, B // 2, 128), jnp.float32),
        mesh=mesh,
        scratch_types=[
            pltpu.VMEM((H, BPW), jnp.int32),
            pltpu.VMEM((2, K, BPW, 32), jnp.uint32),
            pltpu.VMEM((ROWS2, 128), jnp.float32),
            pltpu.SemaphoreType.DMA,
            pltpu.SemaphoreType.DMA,
        ],
        compiler_params=pltpu.CompilerParams(use_tc_tiling_on_sc=False,
                                             needs_layout_passes=False),
    )(xt, packed)


def _bn_dot_body(emb_ref, gamma_ref, beta_ref, out_ref):
    gamma = gamma_ref[...]
    beta = beta_ref[...]

    def bn(h):
        n = 2.0 / B
        m = jnp.sum(h, axis=0, keepdims=True) * n
        sq = jnp.sum(h * h, axis=0, keepdims=True) * n
        mu = (m[:, :D] + m[:, D:]) * 0.5
        var = (sq[:, :D] + sq[:, D:]) * 0.5 - mu * mu
        a = gamma * lax.rsqrt(var + EPS)
        b = beta - a * mu
        a2 = jnp.concatenate([a, a], axis=1)
        b2 = jnp.concatenate([b, b], axis=1)
        return h * a2 + b2

    u = bn(emb_ref[0] * (1.0 / H))
    g = bn(emb_ref[1] * (1.0 / H))
    prod = u * g
    z0 = jnp.sum(prod[:, :D], axis=1, keepdims=True)
    z1 = jnp.sum(prod[:, D:], axis=1, keepdims=True)
    out_ref[...] = jax.nn.sigmoid(jnp.concatenate([z0, z1], axis=1))


def _bn_dot(pooled, gamma, beta):
    return pl.pallas_call(
        _bn_dot_body,
        out_shape=jax.ShapeDtypeStruct((B // 2, 2), jnp.float32),
    )(pooled, gamma, beta)


def _pack_u32(table):
    def bf16top(x):
        u = lax.bitcast_convert_type(x, jnp.uint32)
        r = u + jnp.uint32(0x7FFF) + ((u >> 16) & jnp.uint32(1))
        return r & jnp.uint32(0xFFFF0000)
    return (bf16top(table[:, :32]) >> 16) | bf16top(table[:, 32:])


def kernel(x, user_table, genre_table, gamma, beta):
    zero = lax.optimization_barrier(jnp.int32(0))
    xt = (jnp.transpose(x.astype(jnp.int32), (0, 2, 1))
          .reshape(2, H, NW, 128) ^ zero)
    packed = lax.optimization_barrier(
        jnp.concatenate([_pack_u32(user_table), _pack_u32(genre_table)]))
    pooled = _pool(xt, packed)
    perm = jnp.asarray(_PERM)
    z = _bn_dot(pooled, gamma[perm].reshape(1, D), beta[perm].reshape(1, D))
    return z.reshape(B)


# speedup vs baseline: 1.8712x; 1.0427x over previous
import jax
import jax.numpy as jnp
from jax import lax
from jax.experimental import pallas as pl
from jax.experimental.pallas import tpu as pltpu
from jax.experimental.pallas import tpu_sc as plsc

NC = 2
NS = 16
LANES = 16
NW = NC * NS

B = 4096
H = 50
D = 64
BPW = B // NW
ROWS2 = BPW // 2
EPS = 1e-5

K = 6
NGRP = (H + K - 1) // K


def _grp_hs(g):
    return range(g * K, min((g + 1) * K, H))


def _pool_body(xt_hbm, user_hbm, genre_hbm, out_hbm,
               idx_v, bufs, acc, sem0, sem1):
    wid = lax.axis_index("s") * NC + lax.axis_index("c")
    sems = (sem0, sem1)

    for t, table in ((0, user_hbm), (1, genre_hbm)):
        pltpu.sync_copy(xt_hbm.at[t, :, wid], idx_v)

        def _issue(g):
            bank = g % 2
            for k, h in enumerate(_grp_hs(g)):
                pltpu.async_copy(table.at[idx_v.at[h]], bufs.at[bank, k],
                                 sems[bank])

        def _drain(g):
            bank = g % 2
            for k, h in enumerate(_grp_hs(g)):
                pltpu.make_async_copy(table.at[idx_v.at[h]], bufs.at[bank, k],
                                      sems[bank]).wait()

        def _accum(g):
            bank = g % 2
            nk = len(_grp_hs(g))

            @plsc.parallel_loop(0, ROWS2, unroll=2)
            def body(p):
                for jj in range(8):
                    i = 2 * p + jj // 4
                    sub = (jj % 4) * LANES
                    s = bufs[bank, 0, i, pl.ds(sub, LANES)]
                    for k in range(1, nk):
                        s = s + bufs[bank, k, i, pl.ds(sub, LANES)]
                    if g > 0:
                        s = s + acc[p, pl.ds(jj * LANES, LANES)]
                    acc[p, pl.ds(jj * LANES, LANES)] = s

        _issue(0)
        _issue(1)
        for g in range(NGRP):
            _drain(g)
            _accum(g)
            if g + 2 < NGRP:
                _issue(g + 2)

        pltpu.sync_copy(acc, out_hbm.at[t, pl.ds(wid * ROWS2, ROWS2)])


def _pool(xt, user_table, genre_table):
    mesh = plsc.VectorSubcoreMesh(core_axis_name="c", subcore_axis_name="s",
                                  num_cores=NC, num_subcores=NS)
    return pl.kernel(
        _pool_body,
        out_type=jax.ShapeDtypeStruct((2, B // 2, 128), jnp.float32),
        mesh=mesh,
        scratch_types=[
            pltpu.VMEM((H, BPW), jnp.int32),
            pltpu.VMEM((2, K, BPW, D), jnp.float32),
            pltpu.VMEM((ROWS2, 128), jnp.float32),
            pltpu.SemaphoreType.DMA,
            pltpu.SemaphoreType.DMA,
        ],
        compiler_params=pltpu.CompilerParams(use_tc_tiling_on_sc=False),
    )(xt, user_table, genre_table)


def _bn_dot_body(emb_ref, gamma_ref, beta_ref, out_ref):
    gamma = gamma_ref[...]
    beta = beta_ref[...]

    def bn(h):
        n = 2.0 / B
        m = jnp.sum(h, axis=0, keepdims=True) * n
        sq = jnp.sum(h * h, axis=0, keepdims=True) * n
        mu = (m[:, :D] + m[:, D:]) * 0.5
        var = (sq[:, :D] + sq[:, D:]) * 0.5 - mu * mu
        a = gamma * lax.rsqrt(var + EPS)
        b = beta - a * mu
        a2 = jnp.concatenate([a, a], axis=1)
        b2 = jnp.concatenate([b, b], axis=1)
        return h * a2 + b2

    u = bn(emb_ref[0] * (1.0 / H))
    g = bn(emb_ref[1] * (1.0 / H))
    prod = u * g
    z0 = jnp.sum(prod[:, :D], axis=1, keepdims=True)
    z1 = jnp.sum(prod[:, D:], axis=1, keepdims=True)
    out_ref[...] = jax.nn.sigmoid(jnp.concatenate([z0, z1], axis=1))


def _bn_dot(pooled, gamma, beta):
    return pl.pallas_call(
        _bn_dot_body,
        out_shape=jax.ShapeDtypeStruct((B // 2, 2), jnp.float32),
    )(pooled, gamma, beta)


def kernel(x, user_table, genre_table, gamma, beta):
    xt = jnp.transpose(x.astype(jnp.int32), (0, 2, 1)).reshape(2, H, NW, 128)
    pooled = _pool(xt, user_table, genre_table)
    z = _bn_dot(pooled, gamma.reshape(1, D), beta.reshape(1, D))
    return z.reshape(B)
